# Initial kernel scaffold; baseline (speedup 1.0000x reference)
import jax
import jax.numpy as jnp
from jax.experimental import pallas as pl


def kernel(x, params):
    raise NotImplementedError("write your pallas kernel here")


# speedup vs baseline: 1.7261x; 1.7261x over previous
import numpy as np
import jax
import jax.numpy as jnp
from jax.experimental import pallas as pl
from jax.experimental.pallas import tpu as pltpu

_EPS = 1e-5
_HEAD_G = 3
_CAT_CAP = 13 * 2 ** 20
_F32 = jnp.float32
_DOT_DIMS = (((1,), (0,)), ((), ()))


def _fold(w_oihw, bn=None, bias=None):
    w = jnp.transpose(w_oihw, (2, 3, 1, 0)).astype(_F32)
    o = w.shape[-1]
    b = jnp.zeros((o,), _F32) if bias is None else bias.astype(_F32)
    if bn is not None:
        s = bn['g'] * jax.lax.rsqrt(bn['v'] + _EPS)
        w = w * s
        b = (b - bn['m']) * s + bn['b']
    return w, b


def _dot(a, b):
    return jax.lax.dot_general(a, b, _DOT_DIMS, preferred_element_type=_F32)


def _group(starts):
    by_sx = {}
    for idx, (r, c) in enumerate(starts):
        by_sx.setdefault(c, []).append((r, idx))
    order, groups = [], []
    start = 0
    for c in sorted(by_sx):
        ent = sorted(by_sx[c])
        rows = [r for r, _ in ent]
        assert rows == list(range(rows[0], rows[0] + len(rows)))
        groups.append((c, start, len(ent), rows[0]))
        order.extend(idx for _, idx in ent)
        start += len(ent)
    return order, groups


def _tap_conv(xs_pad, wstacks, groups, bias, ho, wo, *, relu=False,
              res=None):
    n_in = len(xs_pad)
    b_dim = xs_pad[0].shape[0]
    co = wstacks[0].shape[-1]
    hw = ho * wo
    cis = [x.shape[-1] for x in xs_pad]

    def body(*refs):
        x_refs = refs[:n_in]
        w_refs = refs[n_in:2 * n_in]
        b_ref = refs[2 * n_in]
        idx = 2 * n_in + 1
        res_ref = None
        if res is not None:
            res_ref = refs[idx]
            idx += 1
        out_ref = refs[idx]
        acc_ref = refs[idx + 1]
        acc_ref[...] = jnp.broadcast_to(b_ref[0], (hw, co))
        for i in range(n_in):
            if groups[i][0] == 'rowcat':
                _, cols, row0, nrows = groups[i]

                def tap(j, _, x_r=x_refs[i], w_r=w_refs[i], ci=cis[i],
                        cols=cols, row0=row0):
                    parts = [x_r[0, pl.ds(row0 + j, ho), c:c + wo, :]
                             for c in cols]
                    sl = (parts[0] if len(parts) == 1
                          else jnp.concatenate(parts, axis=-1))
                    wt = w_r[pl.ds(j, 1)][0]
                    acc_ref[...] = acc_ref[...] + _dot(
                        sl.reshape(hw, len(cols) * ci), wt)
                    return 0

                jax.lax.fori_loop(0, nrows, tap, 0)
            else:
                for (sx, wst, cnt, row0) in groups[i][1]:

                    def tap(j, _, x_r=x_refs[i], w_r=w_refs[i], ci=cis[i],
                            sx=sx, wst=wst, row0=row0):
                        sl = x_r[0, pl.ds(row0 + j, ho), sx:sx + wo, :]
                        wt = w_r[pl.ds(wst + j, 1)][0]
                        acc_ref[...] = acc_ref[...] + _dot(
                            sl.reshape(hw, ci), wt)
                        return 0

                    jax.lax.fori_loop(0, cnt, tap, 0)
        v = acc_ref[...]
        if res_ref is not None:
            v = v + res_ref[0].reshape(hw, co)
        if relu:
            v = jnp.maximum(v, 0.0)
        out_ref[0] = v.reshape(ho, wo, co)

    in_specs = []
    for x in xs_pad:
        hp, wp, ci = x.shape[1], x.shape[2], x.shape[3]
        in_specs.append(pl.BlockSpec((1, hp, wp, ci), lambda b: (b, 0, 0, 0)))
    for w in wstacks:
        in_specs.append(pl.BlockSpec(w.shape, lambda b: (0, 0, 0)))
    in_specs.append(pl.BlockSpec((1, co), lambda b: (0, 0)))
    args = list(xs_pad) + list(wstacks) + [bias.reshape(1, co)]
    if res is not None:
        in_specs.append(pl.BlockSpec((1, ho, wo, co), lambda b: (b, 0, 0, 0)))
        args.append(res)
    return pl.pallas_call(
        body,
        grid=(b_dim,),
        in_specs=in_specs,
        out_specs=pl.BlockSpec((1, ho, wo, co), lambda b: (b, 0, 0, 0)),
        out_shape=jax.ShapeDtypeStruct((b_dim, ho, wo, co), _F32),
        scratch_shapes=[pltpu.VMEM((hw, co), _F32)],
        compiler_params=pltpu.CompilerParams(
            dimension_semantics=("arbitrary",)),
    )(*args)


def _conv_s1(x, w_hwio, bias, *, relu=False, res=None):
    k = w_hwio.shape[0]
    p = (k - 1) // 2
    ci, co = w_hwio.shape[2], w_hwio.shape[3]
    ho, wo = x.shape[1], x.shape[2]
    xp = jnp.pad(x, ((0, 0), (p, p), (p, p), (0, 0)))
    if ho * wo * k * ci * 4 <= _CAT_CAP:
        wcat = w_hwio.reshape(k, k * ci, co)
        spec = ('rowcat', list(range(k)), 0, k)
        return _tap_conv([xp], [wcat], [spec], bias, ho, wo, relu=relu,
                         res=res)
    order, groups = _group([(dy, dx) for dy in range(k) for dx in range(k)])
    wst = w_hwio.reshape(k * k, ci, co)[np.array(order)]
    return _tap_conv([xp], [wst], [('col', groups)], bias, ho, wo,
                     relu=relu, res=res)


def _conv_s2(x, w_hwio, bias, *, k, pad, relu=False):
    ho, wo = x.shape[1] // 2, x.shape[2] // 2
    taps = {}
    for dy in range(k):
        for dx in range(k):
            oy, ox = dy - pad, dx - pad
            py, sy = (0, oy // 2) if oy % 2 == 0 else (1, (oy - 1) // 2)
            px, sx = (0, ox // 2) if ox % 2 == 0 else (1, (ox - 1) // 2)
            taps.setdefault((py, px), []).append((sy, sx, dy, dx))
    ci = w_hwio.shape[2]
    xs, wstacks, groups_l = [], [], []
    for (py, px) in sorted(taps):
        tl = taps[(py, px)]
        p = max(max(abs(sy), abs(sx)) for (sy, sx, _, _) in tl)
        xph = x[:, py::2, px::2, :]
        xs.append(jnp.pad(xph, ((0, 0), (p, p), (p, p), (0, 0))))
        rows = sorted({sy + p for (sy, _, _, _) in tl})
        cols = sorted({sx + p for (_, sx, _, _) in tl})
        dy_of = {sy + p: dy for (sy, _, dy, _) in tl}
        dx_of = {sx + p: dx for (_, sx, _, dx) in tl}
        assert len(tl) == len(rows) * len(cols)
        assert rows == list(range(rows[0], rows[0] + len(rows)))
        if ho * wo * len(cols) * ci * 4 <= _CAT_CAP:
            wcat = jnp.stack([
                jnp.concatenate([w_hwio[dy_of[r], dx_of[c]] for c in cols],
                                axis=0) for r in rows])
            wstacks.append(wcat)
            groups_l.append(('rowcat', cols, rows[0], len(rows)))
        else:
            order, groups = _group([(sy + p, sx + p)
                                    for (sy, sx, _, _) in tl])
            wst = jnp.stack([w_hwio[dy, dx] for (_, _, dy, dx) in tl])
            wstacks.append(wst[np.array(order)])
            groups_l.append(('col', groups))
    return _tap_conv(xs, wstacks, groups_l, bias, ho, wo, relu=relu)


def _mm(x2, w, bias, *, relu=False):
    m, k = x2.shape
    n = w.shape[-1]
    mb = min(m, 4096)
    nb = min(n, 2048)

    def body(x_ref, w_ref, b_ref, o_ref):
        acc = _dot(x_ref[...], w_ref[...]) + b_ref[0]
        if relu:
            acc = jnp.maximum(acc, 0.0)
        o_ref[...] = acc

    return pl.pallas_call(
        body,
        grid=(m // mb, n // nb),
        in_specs=[
            pl.BlockSpec((mb, k), lambda i, j: (i, 0)),
            pl.BlockSpec((k, nb), lambda i, j: (0, j)),
            pl.BlockSpec((1, nb), lambda i, j: (0, j)),
        ],
        out_specs=pl.BlockSpec((mb, nb), lambda i, j: (i, j)),
        out_shape=jax.ShapeDtypeStruct((m, n), _F32),
        compiler_params=pltpu.CompilerParams(
            dimension_semantics=("arbitrary", "arbitrary")),
    )(x2, w, bias.reshape(1, n))


_S3 = [(dy - 1, dx - 1) for dy in range(3) for dx in range(3)]


def _heads_fused(feat_pad, w1g, b1g, w2g, b2g, ho, wo):
    b_dim, _, _, ci = feat_pad.shape
    ng, _, _, cg = w1g.shape
    t = w2g.shape[1]
    cm = w2g.shape[-1]
    hw = ho * wo

    def body(f_ref, w1_ref, b1_ref, w2_ref, b2_ref, o_ref, h1p, acc2):
        h1p[...] = jnp.zeros(h1p.shape, _F32)
        h1p[1:1 + ho, 8:8 + wo, :] = jnp.broadcast_to(
            b1_ref[0, 0], (ho, wo, cg))

        def tap1(j, _):
            sl = jnp.concatenate(
                [f_ref[0, pl.ds(j, ho), c:c + wo, :] for c in (0, 1, 2)],
                axis=-1)
            wt = w1_ref[0, pl.ds(j, 1)][0]
            h1p[1:1 + ho, 8:8 + wo, :] = (
                h1p[1:1 + ho, 8:8 + wo, :]
                + _dot(sl.reshape(hw, 3 * ci), wt).reshape(ho, wo, cg))
            return 0

        jax.lax.fori_loop(0, 3, tap1, 0)
        h1p[1:1 + ho, 8:8 + wo, :] = jnp.maximum(
            h1p[1:1 + ho, 8:8 + wo, :], 0.0)
        acc2[...] = jnp.broadcast_to(b2_ref[0, 0], (hw, cm))
        for dx in range(3):

            def tap2(j, _, dx=dx):
                sl = h1p[pl.ds(j, ho), 7 + dx:7 + dx + wo, :]
                wt = w2_ref[0, pl.ds(dx * 3 + j, 1)][0]
                acc2[...] = acc2[...] + _dot(sl.reshape(hw, cg), wt)
                return 0

            jax.lax.fori_loop(0, 3, tap2, 0)
        o_ref[0, 0] = jnp.transpose(acc2[...], (1, 0)).reshape(cm, ho, wo)

    return pl.pallas_call(
        body,
        grid=(b_dim, ng),
        in_specs=[
            pl.BlockSpec((1, ho + 2, wo + 2, ci),
                         lambda b, g: (b, 0, 0, 0)),
            pl.BlockSpec((1, 3, 3 * ci, cg), lambda b, g: (g, 0, 0, 0)),
            pl.BlockSpec((1, 1, cg), lambda b, g: (g, 0, 0)),
            pl.BlockSpec((1, t, cg, cm), lambda b, g: (g, 0, 0, 0)),
            pl.BlockSpec((1, 1, cm), lambda b, g: (g, 0, 0)),
        ],
        out_specs=pl.BlockSpec((1, 1, cm, ho, wo),
                               lambda b, g: (b, g, 0, 0, 0)),
        out_shape=jax.ShapeDtypeStruct((b_dim, ng, cm, ho, wo), _F32),
        scratch_shapes=[
            pltpu.VMEM((ho + 2, wo + 16, cg), _F32),
            pltpu.VMEM((hw, cm), _F32),
        ],
        compiler_params=pltpu.CompilerParams(
            dimension_semantics=("arbitrary", "arbitrary")),
    )(feat_pad, w1g, b1g, w2g, b2g)


def _pixel_shuffle(u, b_dim, hs, ws, k, o):
    u = u.reshape(b_dim, hs, ws, k, k, o)
    u = jnp.transpose(u, (0, 1, 3, 2, 4, 5))
    return u.reshape(b_dim, hs * k, ws * k, o)


def kernel(x, params):
    head_order = [('reg', 2), ('height', 1), ('dim', 3), ('rot', 2),
                  ('vel', 2)]
    num_classes = [1, 2, 2, 1, 2, 2]
    b_dim = x.shape[0]
    xh = jnp.transpose(x, (0, 2, 3, 1))

    w, b = _fold(params['stem_w'], params['stem_bn'])
    y = _conv_s2(xh, w, b, k=7, pad=3, relu=True)

    strides = [1, 2, 2]
    trunk_outs = []
    for si, blocks in enumerate(params['stages']):
        for bi, blk in enumerate(blocks):
            s = strides[si] if bi == 0 else 1
            w1, b1 = _fold(blk['w1'], blk['bn1'])
            w2, b2 = _fold(blk['w2'], blk['bn2'])
            if 'down_w' in blk:
                wd, bd = _fold(blk['down_w'], blk['down_bn'])
                wd = wd.reshape(1, wd.shape[2], wd.shape[3])
                src = y[:, ::2, ::2, :] if s == 2 else y
                idt = _tap_conv([src], [wd], [('rowcat', [0], 0, 1)], bd,
                                src.shape[1], src.shape[2])
            else:
                idt = y
            if s == 2:
                h = _conv_s2(y, w1, b1, k=3, pad=1, relu=True)
            else:
                h = _conv_s1(y, w1, b1, relu=True)
            y = _conv_s1(h, w2, b2, res=idt, relu=True)
        trunk_outs.append(y)

    ups = []
    for tr, nk in zip(trunk_outs, params['neck']):
        ic, oc, us, _ = nk['w'].shape
        s = nk['bn']['g'] * jax.lax.rsqrt(nk['bn']['v'] + _EPS)
        wf = jnp.transpose(nk['w'], (0, 2, 3, 1)) * s
        bf = jnp.broadcast_to((nk['bn']['b'] - nk['bn']['m'] * s)
                              [None, None, :], (us, us, oc)).reshape(-1)
        hs, ws = tr.shape[1], tr.shape[2]
        u = _mm(tr.reshape(b_dim * hs * ws, ic),
                wf.reshape(ic, us * us * oc), bf, relu=True)
        ups.append(_pixel_shuffle(u, b_dim, hs, ws, us, oc))

    wsh, bsh = _fold(params['shared_w'], params['shared_bn'])
    order9, groups9 = _group([(dy, dx) for dy in range(3)
                              for dx in range(3)])
    off = 0
    feat = None
    for li, u in enumerate(ups):
        ci = u.shape[-1]
        up = jnp.pad(u, ((0, 0), (1, 1), (1, 1), (0, 0)))
        wslice = wsh[:, :, off:off + ci, :]
        bias_i = bsh if li == 0 else jnp.zeros((64,), _F32)
        if 128 * 128 * 3 * ci * 4 <= _CAT_CAP:
            wst = wslice.reshape(3, 3 * ci, 64)
            spec = ('rowcat', [0, 1, 2], 0, 3)
        else:
            wst = wslice.reshape(9, ci, 64)[np.array(order9)]
            spec = ('col', groups9)
        feat = _tap_conv([up], [wst], [spec], bias_i, 128, 128,
                         relu=(li == len(ups) - 1), res=feat)
        off += ci

    heads_flat = []
    for heads, nc in zip(params['tasks'], num_classes):
        for name, c in head_order + [('heatmap', nc)]:
            hp = heads[name]
            hw1, hb1 = _fold(hp['w1'], hp['bn1'])
            hw2 = jnp.transpose(hp['w2'], (2, 3, 1, 0))
            heads_flat.append((hw1, hb1, hw2, hp['b2'], c))
    groups = [heads_flat[i:i + _HEAD_G]
              for i in range(0, len(heads_flat), _HEAD_G)]
    group_cs = [sum(h[4] for h in g) for g in groups]
    cm = max(group_cs)
    cg = 64 * _HEAD_G
    w1l, b1l, w2l, b2l = [], [], [], []
    for g in groups:
        w1l.append(jnp.concatenate([h[0] for h in g], axis=-1)
                   .reshape(3, 192, cg))
        b1l.append(jnp.concatenate([h[1] for h in g]))
        w2blk = jnp.zeros((9, cg, cm), _F32)
        coff = 0
        for j, h in enumerate(g):
            c = h[4]
            w2blk = w2blk.at[:, j * 64:(j + 1) * 64, coff:coff + c].set(
                h[2].reshape(9, 64, c))
            coff += c
        w2l.append(w2blk)
        b2l.append(jnp.concatenate(
            [h[3] for h in g] + [jnp.zeros((cm - coff,), _F32)]))
    feat_pad = jnp.pad(feat, ((0, 0), (1, 1), (1, 1), (0, 0)))
    dxmaj = np.array([dy * 3 + dx for dx in range(3) for dy in range(3)])
    out5 = _heads_fused(feat_pad,
                        jnp.stack(w1l),
                        jnp.stack(b1l).reshape(-1, 1, cg),
                        jnp.stack(w2l)[:, dxmaj],
                        jnp.stack(b2l).reshape(-1, 1, cm),
                        128, 128)
    parts = [out5[:, g, :c] for g, c in enumerate(group_cs)]
    return jnp.concatenate(parts, axis=1)


# speedup vs baseline: 2.3740x; 1.3754x over previous
import numpy as np
import jax
import jax.numpy as jnp
from jax.experimental import pallas as pl
from jax.experimental.pallas import tpu as pltpu

_EPS = 1e-5
_HEAD_G = 4
_CAT_CAP = 13 * 2 ** 20
_F32 = jnp.float32
_DOT_DIMS = (((1,), (0,)), ((), ()))


def _fold(w_oihw, bn=None, bias=None):
    w = jnp.transpose(w_oihw, (2, 3, 1, 0)).astype(_F32)
    o = w.shape[-1]
    b = jnp.zeros((o,), _F32) if bias is None else bias.astype(_F32)
    if bn is not None:
        s = bn['g'] * jax.lax.rsqrt(bn['v'] + _EPS)
        w = w * s
        b = (b - bn['m']) * s + bn['b']
    return w, b


def _dot(a, b):
    return jax.lax.dot_general(a, b, _DOT_DIMS, preferred_element_type=_F32)


def _group(starts):
    by_sx = {}
    for idx, (r, c) in enumerate(starts):
        by_sx.setdefault(c, []).append((r, idx))
    order, groups = [], []
    start = 0
    for c in sorted(by_sx):
        ent = sorted(by_sx[c])
        rows = [r for r, _ in ent]
        assert rows == list(range(rows[0], rows[0] + len(rows)))
        groups.append((c, start, len(ent), rows[0]))
        order.extend(idx for _, idx in ent)
        start += len(ent)
    return order, groups


def _tap_conv(xs_pad, wstacks, groups, bias, ho, wo, *, relu=False,
              res=None):
    n_in = len(xs_pad)
    b_dim = xs_pad[0].shape[0]
    co = wstacks[0].shape[-1]
    hw = ho * wo
    cis = [x.shape[-1] for x in xs_pad]

    def body(*refs):
        x_refs = refs[:n_in]
        w_refs = refs[n_in:2 * n_in]
        b_ref = refs[2 * n_in]
        idx = 2 * n_in + 1
        res_ref = None
        if res is not None:
            res_ref = refs[idx]
            idx += 1
        out_ref = refs[idx]
        acc_ref = refs[idx + 1]
        acc_ref[...] = jnp.broadcast_to(b_ref[0], (hw, co))
        for i in range(n_in):
            if groups[i][0] == 'rowcat':
                _, cols, row0, nrows = groups[i]

                def tap(j, _, x_r=x_refs[i], w_r=w_refs[i], ci=cis[i],
                        cols=cols, row0=row0):
                    parts = [x_r[0, pl.ds(row0 + j, ho), c:c + wo, :]
                             for c in cols]
                    sl = (parts[0] if len(parts) == 1
                          else jnp.concatenate(parts, axis=-1))
                    wt = w_r[pl.ds(j, 1)][0]
                    acc_ref[...] = acc_ref[...] + _dot(
                        sl.reshape(hw, len(cols) * ci), wt)
                    return 0

                jax.lax.fori_loop(0, nrows, tap, 0)
            else:
                for (sx, wst, cnt, row0) in groups[i][1]:

                    def tap(j, _, x_r=x_refs[i], w_r=w_refs[i], ci=cis[i],
                            sx=sx, wst=wst, row0=row0):
                        sl = x_r[0, pl.ds(row0 + j, ho), sx:sx + wo, :]
                        wt = w_r[pl.ds(wst + j, 1)][0]
                        acc_ref[...] = acc_ref[...] + _dot(
                            sl.reshape(hw, ci), wt)
                        return 0

                    jax.lax.fori_loop(0, cnt, tap, 0)
        v = acc_ref[...]
        if res_ref is not None:
            v = v + res_ref[0].reshape(hw, co)
        if relu:
            v = jnp.maximum(v, 0.0)
        out_ref[0] = v.reshape(ho, wo, co)

    in_specs = []
    for x in xs_pad:
        hp, wp, ci = x.shape[1], x.shape[2], x.shape[3]
        in_specs.append(pl.BlockSpec((1, hp, wp, ci), lambda b: (b, 0, 0, 0)))
    for w in wstacks:
        in_specs.append(pl.BlockSpec(w.shape, lambda b: (0, 0, 0)))
    in_specs.append(pl.BlockSpec((1, co), lambda b: (0, 0)))
    args = list(xs_pad) + list(wstacks) + [bias.reshape(1, co)]
    if res is not None:
        in_specs.append(pl.BlockSpec((1, ho, wo, co), lambda b: (b, 0, 0, 0)))
        args.append(res)
    return pl.pallas_call(
        body,
        grid=(b_dim,),
        in_specs=in_specs,
        out_specs=pl.BlockSpec((1, ho, wo, co), lambda b: (b, 0, 0, 0)),
        out_shape=jax.ShapeDtypeStruct((b_dim, ho, wo, co), _F32),
        scratch_shapes=[pltpu.VMEM((hw, co), _F32)],
        compiler_params=pltpu.CompilerParams(
            dimension_semantics=("arbitrary",)),
    )(*args)


def _conv_s1(x, w_hwio, bias, *, relu=False, res=None):
    k = w_hwio.shape[0]
    p = (k - 1) // 2
    ci, co = w_hwio.shape[2], w_hwio.shape[3]
    ho, wo = x.shape[1], x.shape[2]
    xp = jnp.pad(x, ((0, 0), (p, p), (p, p), (0, 0)))
    if ho * wo * k * ci * 4 <= _CAT_CAP:
        wcat = w_hwio.reshape(k, k * ci, co)
        spec = ('rowcat', list(range(k)), 0, k)
        return _tap_conv([xp], [wcat], [spec], bias, ho, wo, relu=relu,
                         res=res)
    order, groups = _group([(dy, dx) for dy in range(k) for dx in range(k)])
    wst = w_hwio.reshape(k * k, ci, co)[np.array(order)]
    return _tap_conv([xp], [wst], [('col', groups)], bias, ho, wo,
                     relu=relu, res=res)


def _conv_s2(x, w_hwio, bias, *, k, pad, relu=False):
    ho, wo = x.shape[1] // 2, x.shape[2] // 2
    taps = {}
    for dy in range(k):
        for dx in range(k):
            oy, ox = dy - pad, dx - pad
            py, sy = (0, oy // 2) if oy % 2 == 0 else (1, (oy - 1) // 2)
            px, sx = (0, ox // 2) if ox % 2 == 0 else (1, (ox - 1) // 2)
            taps.setdefault((py, px), []).append((sy, sx, dy, dx))
    ci = w_hwio.shape[2]
    xs, wstacks, groups_l = [], [], []
    for (py, px) in sorted(taps):
        tl = taps[(py, px)]
        p = max(max(abs(sy), abs(sx)) for (sy, sx, _, _) in tl)
        xph = x[:, py::2, px::2, :]
        xs.append(jnp.pad(xph, ((0, 0), (p, p), (p, p), (0, 0))))
        rows = sorted({sy + p for (sy, _, _, _) in tl})
        cols = sorted({sx + p for (_, sx, _, _) in tl})
        dy_of = {sy + p: dy for (sy, _, dy, _) in tl}
        dx_of = {sx + p: dx for (_, sx, _, dx) in tl}
        assert len(tl) == len(rows) * len(cols)
        assert rows == list(range(rows[0], rows[0] + len(rows)))
        if ho * wo * len(cols) * ci * 4 <= _CAT_CAP:
            wcat = jnp.stack([
                jnp.concatenate([w_hwio[dy_of[r], dx_of[c]] for c in cols],
                                axis=0) for r in rows])
            wstacks.append(wcat)
            groups_l.append(('rowcat', cols, rows[0], len(rows)))
        else:
            order, groups = _group([(sy + p, sx + p)
                                    for (sy, sx, _, _) in tl])
            wst = jnp.stack([w_hwio[dy, dx] for (_, _, dy, dx) in tl])
            wstacks.append(wst[np.array(order)])
            groups_l.append(('col', groups))
    return _tap_conv(xs, wstacks, groups_l, bias, ho, wo, relu=relu)


def _mm(x2, w, bias, *, relu=False):
    m, k = x2.shape
    n = w.shape[-1]
    mb = min(m, 4096)
    nb = min(n, 2048)

    def body(x_ref, w_ref, b_ref, o_ref):
        acc = _dot(x_ref[...], w_ref[...]) + b_ref[0]
        if relu:
            acc = jnp.maximum(acc, 0.0)
        o_ref[...] = acc

    return pl.pallas_call(
        body,
        grid=(m // mb, n // nb),
        in_specs=[
            pl.BlockSpec((mb, k), lambda i, j: (i, 0)),
            pl.BlockSpec((k, nb), lambda i, j: (0, j)),
            pl.BlockSpec((1, nb), lambda i, j: (0, j)),
        ],
        out_specs=pl.BlockSpec((mb, nb), lambda i, j: (i, j)),
        out_shape=jax.ShapeDtypeStruct((m, n), _F32),
        compiler_params=pltpu.CompilerParams(
            dimension_semantics=("arbitrary", "arbitrary")),
    )(x2, w, bias.reshape(1, n))


_S3 = [(dy - 1, dx - 1) for dy in range(3) for dx in range(3)]


def _heads_fused(feat_pad, w1g, b1g, w2g, b2g, ho, wo):
    b_dim, _, _, ci = feat_pad.shape
    ng, _, _, cg = w1g.shape
    cm = w2g.shape[-1] // 9
    hw = ho * wo

    hp, wp = ho + 3, wo + 16
    flat = hp * wp

    def body(f_ref, w1_ref, b1_ref, w2_ref, b2_ref, o_ref, h1p, ptr,
             acct):
        h1p[...] = jnp.zeros(h1p.shape, _F32)
        h1p[1:1 + ho, 8:8 + wo, :] = jnp.broadcast_to(
            b1_ref[0, 0], (ho, wo, cg))
        for r0 in (0, ho // 2):

            def tap1(j, _, r0=r0):
                sl = jnp.concatenate(
                    [f_ref[0, pl.ds(r0 + j, ho // 2), c:c + wo, :]
                     for c in (0, 1, 2)], axis=-1)
                wt = w1_ref[0, pl.ds(j, 1)][0]
                h1p[1 + r0:1 + r0 + ho // 2, 8:8 + wo, :] = (
                    h1p[1 + r0:1 + r0 + ho // 2, 8:8 + wo, :]
                    + _dot(sl.reshape(ho // 2 * wo, 3 * ci),
                           wt).reshape(ho // 2, wo, cg))
                return 0

            jax.lax.fori_loop(0, 3, tap1, 0)
        h1p[1:1 + ho, 8:8 + wo, :] = jnp.maximum(
            h1p[1:1 + ho, 8:8 + wo, :], 0.0)
        nch = 4
        rch = [(k * hp // nch, (k + 1) * hp // nch) for k in range(nch)]
        for (a, b) in rch:
            pc = _dot(h1p[a:b].reshape((b - a) * wp, cg), w2_ref[0])
            ptr[:, a * wp:b * wp] = jnp.transpose(pc, (1, 0))
        acct[...] = jnp.broadcast_to(jnp.transpose(b2_ref[0], (1, 0)),
                                     (cm, ho * wp))
        for t in range(9):
            dy, dx = t // 3, t % 3
            s = dy * wp + dx + 7
            acct[...] = acct[...] + ptr[t * cm:(t + 1) * cm,
                                        s:s + ho * wp]
        o_ref[0, 0] = acct[...].reshape(cm, ho, wp)[:, :, :wo]

    return pl.pallas_call(
        body,
        grid=(b_dim, ng),
        in_specs=[
            pl.BlockSpec((1, ho + 2, wo + 2, ci),
                         lambda b, g: (b, 0, 0, 0)),
            pl.BlockSpec((1, 3, 3 * ci, cg), lambda b, g: (g, 0, 0, 0)),
            pl.BlockSpec((1, 1, cg), lambda b, g: (g, 0, 0)),
            pl.BlockSpec((1, cg, 9 * cm), lambda b, g: (g, 0, 0)),
            pl.BlockSpec((1, 1, cm), lambda b, g: (g, 0, 0)),
        ],
        out_specs=pl.BlockSpec((1, 1, cm, ho, wo),
                               lambda b, g: (b, g, 0, 0, 0)),
        out_shape=jax.ShapeDtypeStruct((b_dim, ng, cm, ho, wo), _F32),
        scratch_shapes=[
            pltpu.VMEM((hp, wp, cg), _F32),
            pltpu.VMEM((9 * cm, flat), _F32),
            pltpu.VMEM((cm, ho * wp), _F32),
        ],
        compiler_params=pltpu.CompilerParams(
            dimension_semantics=("arbitrary", "arbitrary")),
    )(feat_pad, w1g, b1g, w2g, b2g)


def _pixel_shuffle(u, b_dim, hs, ws, k, o):
    u = u.reshape(b_dim, hs, ws, k, k, o)
    u = jnp.transpose(u, (0, 1, 3, 2, 4, 5))
    return u.reshape(b_dim, hs * k, ws * k, o)


def kernel(x, params):
    head_order = [('reg', 2), ('height', 1), ('dim', 3), ('rot', 2),
                  ('vel', 2)]
    num_classes = [1, 2, 2, 1, 2, 2]
    b_dim = x.shape[0]
    xh = jnp.transpose(x, (0, 2, 3, 1))

    w, b = _fold(params['stem_w'], params['stem_bn'])
    y = _conv_s2(xh, w, b, k=7, pad=3, relu=True)

    strides = [1, 2, 2]
    trunk_outs = []
    for si, blocks in enumerate(params['stages']):
        for bi, blk in enumerate(blocks):
            s = strides[si] if bi == 0 else 1
            w1, b1 = _fold(blk['w1'], blk['bn1'])
            w2, b2 = _fold(blk['w2'], blk['bn2'])
            if 'down_w' in blk:
                wd, bd = _fold(blk['down_w'], blk['down_bn'])
                wd = wd.reshape(1, wd.shape[2], wd.shape[3])
                src = y[:, ::2, ::2, :] if s == 2 else y
                idt = _tap_conv([src], [wd], [('rowcat', [0], 0, 1)], bd,
                                src.shape[1], src.shape[2])
            else:
                idt = y
            if s == 2:
                h = _conv_s2(y, w1, b1, k=3, pad=1, relu=True)
            else:
                h = _conv_s1(y, w1, b1, relu=True)
            y = _conv_s1(h, w2, b2, res=idt, relu=True)
        trunk_outs.append(y)

    ups = []
    for tr, nk in zip(trunk_outs, params['neck']):
        ic, oc, us, _ = nk['w'].shape
        s = nk['bn']['g'] * jax.lax.rsqrt(nk['bn']['v'] + _EPS)
        wf = jnp.transpose(nk['w'], (0, 2, 3, 1)) * s
        bf = jnp.broadcast_to((nk['bn']['b'] - nk['bn']['m'] * s)
                              [None, None, :], (us, us, oc)).reshape(-1)
        hs, ws = tr.shape[1], tr.shape[2]
        u = _mm(tr.reshape(b_dim * hs * ws, ic),
                wf.reshape(ic, us * us * oc), bf, relu=True)
        ups.append(_pixel_shuffle(u, b_dim, hs, ws, us, oc))

    wsh, bsh = _fold(params['shared_w'], params['shared_bn'])
    order9, groups9 = _group([(dy, dx) for dy in range(3)
                              for dx in range(3)])
    off = 0
    feat = None
    for li, u in enumerate(ups):
        ci = u.shape[-1]
        up = jnp.pad(u, ((0, 0), (1, 1), (1, 1), (0, 0)))
        wslice = wsh[:, :, off:off + ci, :]
        bias_i = bsh if li == 0 else jnp.zeros((64,), _F32)
        if 128 * 128 * 3 * ci * 4 <= _CAT_CAP:
            wst = wslice.reshape(3, 3 * ci, 64)
            spec = ('rowcat', [0, 1, 2], 0, 3)
        else:
            wst = wslice.reshape(9, ci, 64)[np.array(order9)]
            spec = ('col', groups9)
        feat = _tap_conv([up], [wst], [spec], bias_i, 128, 128,
                         relu=(li == len(ups) - 1), res=feat)
        off += ci

    heads_flat = []
    for heads, nc in zip(params['tasks'], num_classes):
        for name, c in head_order + [('heatmap', nc)]:
            hp = heads[name]
            hw1, hb1 = _fold(hp['w1'], hp['bn1'])
            hw2 = jnp.transpose(hp['w2'], (2, 3, 1, 0))
            heads_flat.append((hw1, hb1, hw2, hp['b2'], c))
    groups = [heads_flat[i:i + _HEAD_G]
              for i in range(0, len(heads_flat), _HEAD_G)]
    group_cs = [sum(h[4] for h in g) for g in groups]
    cm = max(group_cs)
    cg = 64 * _HEAD_G
    w1l, b1l, w2l, b2l = [], [], [], []
    for g in groups:
        w1l.append(jnp.concatenate([h[0] for h in g], axis=-1)
                   .reshape(3, 192, cg))
        b1l.append(jnp.concatenate([h[1] for h in g]))
        w2blk = jnp.zeros((9, cg, cm), _F32)
        coff = 0
        for j, h in enumerate(g):
            c = h[4]
            w2blk = w2blk.at[:, j * 64:(j + 1) * 64, coff:coff + c].set(
                h[2].reshape(9, 64, c))
            coff += c
        w2l.append(jnp.transpose(w2blk, (1, 0, 2)).reshape(cg, 9 * cm))
        b2l.append(jnp.concatenate(
            [h[3] for h in g] + [jnp.zeros((cm - coff,), _F32)]))
    feat_pad = jnp.pad(feat, ((0, 0), (1, 1), (1, 1), (0, 0)))
    out5 = _heads_fused(feat_pad,
                        jnp.stack(w1l),
                        jnp.stack(b1l).reshape(-1, 1, cg),
                        jnp.stack(w2l),
                        jnp.stack(b2l).reshape(-1, 1, cm),
                        128, 128)
    parts = [out5[:, g, :c] for g, c in enumerate(group_cs)]
    return jnp.concatenate(parts, axis=1)
